# baseline (speedup 1.0000x reference)
import functools

import jax
import jax.numpy as jnp
from jax import lax
from jax.experimental import pallas as pl
from jax.experimental.pallas import tpu as pltpu
from jax.experimental.pallas import tpu_sc as plsc

NC = 2
NS = 16
NW = NC * NS
L = 16
CHUNK = 64
RING = 4
IG = 8


@functools.lru_cache(maxsize=None)
def _make_sc_segsum(N, D, CH, NPAD):
    ROWS_PT = NPAD // NS
    NG = CH // IG
    mesh = plsc.VectorSubcoreMesh(core_axis_name="c", subcore_axis_name="s")

    @functools.partial(
        pl.kernel,
        out_type=(
            jax.ShapeDtypeStruct((NC, NPAD, D), jnp.float32),
            jax.ShapeDtypeStruct((NC, NS, NPAD), jnp.float32),
        ),
        mesh=mesh,
        compiler_params=pltpu.CompilerParams(needs_layout_passes=False),
        scratch_types=(
            pltpu.VMEM_SHARED((NPAD, D), jnp.float32),
            pltpu.VMEM((2, IG, CHUNK), jnp.int32),
            pltpu.VMEM((2, IG, CHUNK), jnp.int32),
            pltpu.VMEM((RING, CHUNK, D), jnp.float32),
            pltpu.VMEM((NPAD,), jnp.float32),
            [pltpu.SemaphoreType.DMA] * RING,
            [pltpu.SemaphoreType.DMA] * RING,
            pltpu.SemaphoreType.DMA,
        ),
    )
    def seg(x_hbm, srcs_hbm, dsts_hbm, zeros_hbm, agg_out, deg_out,
            shared_agg, src_v, dst_v, rows_v, deg_v, sg, ss, si):
        c = lax.axis_index("c")
        s = lax.axis_index("s")
        wid = c * NS + s
        pltpu.sync_copy(zeros_hbm.at[pl.ds(s * ROWS_PT, ROWS_PT)],
                        shared_agg.at[pl.ds(s * ROWS_PT, ROWS_PT)])
        zeros16 = jnp.zeros((L,), jnp.float32)

        def zero_body(i, carry):
            deg_v[pl.ds(i * L, L)] = zeros16
            return carry

        lax.fori_loop(0, NPAD // L, zero_body, 0)
        pltpu.sync_copy(srcs_hbm.at[wid, pl.ds(0, IG)], src_v.at[0])
        pltpu.sync_copy(dsts_hbm.at[wid, pl.ds(0, IG)], dst_v.at[0])
        plsc.subcore_barrier()

        ones = jnp.full((L,), 1.0, jnp.float32)
        for b in range(2):
            pltpu.async_copy(x_hbm.at[src_v.at[0, b]], rows_v.at[b], sg[b])

        def group_body(g, carry):
            slot = lax.rem(g, 2)
            nslot = lax.rem(g + 1, 2)

            for jj in range(IG):
                j = g * IG + jj
                b = jj % RING
                b2 = (jj + 2) % RING
                if jj == 2:
                    @pl.when(g + 1 < NG)
                    def _():
                        pltpu.async_copy(
                            srcs_hbm.at[wid, pl.ds((g + 1) * IG, IG)],
                            src_v.at[nslot], si)
                        pltpu.async_copy(
                            dsts_hbm.at[wid, pl.ds((g + 1) * IG, IG)],
                            dst_v.at[nslot], si)
                if jj == IG - 2:
                    @pl.when(g + 1 < NG)
                    def _():
                        pltpu.make_async_copy(
                            srcs_hbm.at[wid, pl.ds((g + 1) * IG, IG)],
                            src_v.at[nslot], si).wait()
                        pltpu.make_async_copy(
                            dsts_hbm.at[wid, pl.ds((g + 1) * IG, IG)],
                            dst_v.at[nslot], si).wait()

                @pl.when(j >= 2)
                def _():
                    if jj >= 2:
                        pidx = dst_v.at[slot, jj - 2]
                    else:
                        pidx = dst_v.at[nslot, jj - 2 + IG]
                    pltpu.make_async_copy(
                        rows_v.at[b2], shared_agg.at[pidx], ss[b2]).wait()

                @pl.when(j + 2 < CH)
                def _():
                    if jj + 2 < IG:
                        gidx = src_v.at[slot, jj + 2]
                    else:
                        gidx = src_v.at[nslot, jj + 2 - IG]
                    pltpu.async_copy(x_hbm.at[gidx], rows_v.at[b2], sg[b2])

                pltpu.make_async_copy(
                    x_hbm.at[src_v.at[slot, jj]], rows_v.at[b], sg[b]).wait()
                pltpu.async_copy(
                    rows_v.at[b], shared_agg.at[dst_v.at[slot, jj]], ss[b],
                    add=True)
                for k in range(CHUNK // L):
                    d = dst_v[slot, jj, pl.ds(k * L, L)]
                    plsc.addupdate_scatter(deg_v, [d], ones)
            return carry

        lax.fori_loop(0, NG, group_body, 0)
        lslot = (NG - 1) % 2
        for j in (CH - 2, CH - 1):
            b = j % RING
            pidx = dst_v.at[lslot, j - (NG - 1) * IG]
            pltpu.make_async_copy(rows_v.at[b], shared_agg.at[pidx], ss[b]).wait()
        plsc.subcore_barrier()
        pltpu.sync_copy(shared_agg.at[pl.ds(s * ROWS_PT, ROWS_PT)],
                        agg_out.at[c, pl.ds(s * ROWS_PT, ROWS_PT)])
        pltpu.sync_copy(deg_v, deg_out.at[c, s])

    return seg


def _finalize_body(x_ref, agg_ref, deg_ref, wl_ref, bl_ref, wr_ref, o_ref):
    agg = agg_ref[0] + agg_ref[1]
    deg = jnp.sum(deg_ref[...], axis=0)
    mean = agg / jnp.clip(deg, 1.0)[:, None]
    dn = (((1,), (1,)), ((), ()))
    o_ref[...] = (
        lax.dot_general(mean, wl_ref[...], dn, preferred_element_type=jnp.float32)
        + lax.dot_general(x_ref[...], wr_ref[...], dn, preferred_element_type=jnp.float32)
        + bl_ref[...])


def kernel(x_src, x_ref, src_edge_indices, ref_edge_indices, W_l, b_l, W_r):
    N, D = x_src.shape
    E = src_edge_indices.shape[0]
    CH = -(-E // (NW * CHUNK))
    CH = -(-CH // IG) * IG
    EP = NW * CH * CHUNK
    NPAD = -(-(N + 1) // 128) * 128

    def prep(edges):
        pad = EP - E
        src = jnp.concatenate(
            [edges[:, 0], jnp.zeros((pad,), jnp.int32)]).reshape(NW, CH, CHUNK)
        dst = jnp.concatenate(
            [edges[:, 1], jnp.full((pad,), N, jnp.int32)]).reshape(NW, CH, CHUNK)
        return src, dst

    ss, ds = prep(src_edge_indices)
    sr, dr = prep(ref_edge_indices)
    zeros = jnp.zeros((NPAD, D), jnp.float32)

    seg = _make_sc_segsum(N, D, CH, NPAD)
    agg_s, deg_s = seg(x_src, ss, ds, zeros)
    agg_r, deg_r = seg(x_ref, sr, dr, zeros)

    RB = 512
    nb = -(-N // RB)
    fin = pl.pallas_call(
        _finalize_body,
        grid=(nb,),
        in_specs=[
            pl.BlockSpec((RB, D), lambda i: (i, 0)),
            pl.BlockSpec((NC, RB, D), lambda i: (0, i, 0)),
            pl.BlockSpec((NW, RB), lambda i: (0, i)),
            pl.BlockSpec((D, D), lambda i: (0, 0)),
            pl.BlockSpec((1, D), lambda i: (0, 0)),
            pl.BlockSpec((D, D), lambda i: (0, 0)),
        ],
        out_specs=pl.BlockSpec((RB, D), lambda i: (i, 0)),
        out_shape=jax.ShapeDtypeStruct((N, D), jnp.float32),
    )
    bl2 = b_l.reshape(1, D)
    out_src = fin(x_src, agg_s, deg_s.reshape(NW, NPAD), W_l, bl2, W_r)
    out_ref = fin(x_ref, agg_r, deg_r.reshape(NW, NPAD), W_l, bl2, W_r)
    return out_src, out_ref


# speedup vs baseline: 1.4067x; 1.4067x over previous
import functools

import jax
import jax.numpy as jnp
from jax import lax
from jax.experimental import pallas as pl
from jax.experimental.pallas import tpu as pltpu
from jax.experimental.pallas import tpu_sc as plsc

NC = 2
NS = 16
L = 16
CHUNK = 64
RING = 4
IG = 8


@functools.lru_cache(maxsize=None)
def _make_sc_segsum(N, D, CH, NPAD):
    ROWS_PT = NPAD // NS
    NG = CH // IG
    mesh = plsc.VectorSubcoreMesh(core_axis_name="c", subcore_axis_name="s")

    @functools.partial(
        pl.kernel,
        out_type=(
            jax.ShapeDtypeStruct((NC, NPAD, D), jnp.float32),
            jax.ShapeDtypeStruct((NC, NS, NPAD), jnp.float32),
        ),
        mesh=mesh,
        compiler_params=pltpu.CompilerParams(needs_layout_passes=False),
        scratch_types=(
            pltpu.VMEM_SHARED((NPAD, D), jnp.float32),
            pltpu.VMEM((2, IG, CHUNK), jnp.int32),
            pltpu.VMEM((2, IG, CHUNK), jnp.int32),
            pltpu.VMEM((RING, CHUNK, D), jnp.float32),
            pltpu.VMEM((NPAD,), jnp.float32),
            [pltpu.SemaphoreType.DMA] * RING,
            [pltpu.SemaphoreType.DMA] * RING,
            pltpu.SemaphoreType.DMA,
        ),
    )
    def seg(xs_hbm, srcs_hbm, dsts_hbm, zeros_hbm, agg_out, deg_out,
            shared_agg, src_v, dst_v, rows_v, deg_v, sg, ss, si):
        c = lax.axis_index("c")
        s = lax.axis_index("s")
        x_hbm = xs_hbm.at[c]
        pltpu.sync_copy(zeros_hbm.at[pl.ds(s * ROWS_PT, ROWS_PT)],
                        shared_agg.at[pl.ds(s * ROWS_PT, ROWS_PT)])
        zeros16 = jnp.zeros((L,), jnp.float32)

        def zero_body(i, carry):
            deg_v[pl.ds(i * L, L)] = zeros16
            return carry

        lax.fori_loop(0, NPAD // L, zero_body, 0)
        pltpu.sync_copy(srcs_hbm.at[c, s, pl.ds(0, IG)], src_v.at[0])
        pltpu.sync_copy(dsts_hbm.at[c, s, pl.ds(0, IG)], dst_v.at[0])
        plsc.subcore_barrier()

        ones = jnp.full((L,), 1.0, jnp.float32)
        for b in range(2):
            pltpu.async_copy(x_hbm.at[src_v.at[0, b]], rows_v.at[b], sg[b])

        def group_body(g, carry):
            slot = lax.rem(g, 2)
            nslot = lax.rem(g + 1, 2)

            for jj in range(IG):
                j = g * IG + jj
                b = jj % RING
                b2 = (jj + 2) % RING
                if jj == 2:
                    @pl.when(g + 1 < NG)
                    def _():
                        pltpu.async_copy(
                            srcs_hbm.at[c, s, pl.ds((g + 1) * IG, IG)],
                            src_v.at[nslot], si)
                        pltpu.async_copy(
                            dsts_hbm.at[c, s, pl.ds((g + 1) * IG, IG)],
                            dst_v.at[nslot], si)
                if jj == IG - 2:
                    @pl.when(g + 1 < NG)
                    def _():
                        pltpu.make_async_copy(
                            srcs_hbm.at[c, s, pl.ds((g + 1) * IG, IG)],
                            src_v.at[nslot], si).wait()
                        pltpu.make_async_copy(
                            dsts_hbm.at[c, s, pl.ds((g + 1) * IG, IG)],
                            dst_v.at[nslot], si).wait()

                @pl.when(j >= 2)
                def _():
                    if jj >= 2:
                        pidx = dst_v.at[slot, jj - 2]
                    else:
                        pidx = dst_v.at[nslot, jj - 2 + IG]
                    pltpu.make_async_copy(
                        rows_v.at[b2], shared_agg.at[pidx], ss[b2]).wait()

                @pl.when(j + 2 < CH)
                def _():
                    if jj + 2 < IG:
                        gidx = src_v.at[slot, jj + 2]
                    else:
                        gidx = src_v.at[nslot, jj + 2 - IG]
                    pltpu.async_copy(x_hbm.at[gidx], rows_v.at[b2], sg[b2])

                pltpu.make_async_copy(
                    x_hbm.at[src_v.at[slot, jj]], rows_v.at[b], sg[b]).wait()
                pltpu.async_copy(
                    rows_v.at[b], shared_agg.at[dst_v.at[slot, jj]], ss[b],
                    add=True)
                for k in range(CHUNK // L):
                    d = dst_v[slot, jj, pl.ds(k * L, L)]
                    plsc.addupdate_scatter(deg_v, [d], ones)
            return carry

        lax.fori_loop(0, NG, group_body, 0)
        lslot = (NG - 1) % 2
        for j in (CH - 2, CH - 1):
            b = j % RING
            pidx = dst_v.at[lslot, j - (NG - 1) * IG]
            pltpu.make_async_copy(rows_v.at[b], shared_agg.at[pidx], ss[b]).wait()
        plsc.subcore_barrier()
        pltpu.sync_copy(shared_agg.at[pl.ds(s * ROWS_PT, ROWS_PT)],
                        agg_out.at[c, pl.ds(s * ROWS_PT, ROWS_PT)])
        pltpu.sync_copy(deg_v, deg_out.at[c, s])

    return seg


def _finalize_body(x_ref, agg_ref, deg_ref, wl_ref, bl_ref, wr_ref, o_ref):
    deg = jnp.sum(deg_ref[...], axis=0)
    mean = agg_ref[...] / jnp.clip(deg, 1.0)[:, None]
    dn = (((1,), (1,)), ((), ()))
    o_ref[...] = (
        lax.dot_general(mean, wl_ref[...], dn, preferred_element_type=jnp.float32)
        + lax.dot_general(x_ref[...], wr_ref[...], dn, preferred_element_type=jnp.float32)
        + bl_ref[...])


def kernel(x_src, x_ref, src_edge_indices, ref_edge_indices, W_l, b_l, W_r):
    N, D = x_src.shape
    E = src_edge_indices.shape[0]
    CH = -(-E // (NS * CHUNK))
    CH = -(-CH // IG) * IG
    EP = NS * CH * CHUNK
    NPAD = -(-(N + 1) // 128) * 128

    def prep(edges):
        pad = EP - E
        src = jnp.concatenate(
            [edges[:, 0], jnp.zeros((pad,), jnp.int32)]).reshape(NS, CH, CHUNK)
        dst = jnp.concatenate(
            [edges[:, 1], jnp.full((pad,), N, jnp.int32)]).reshape(NS, CH, CHUNK)
        return src, dst

    s0, d0 = prep(src_edge_indices)
    s1, d1 = prep(ref_edge_indices)
    xs = jnp.stack([x_src, x_ref])
    srcs = jnp.stack([s0, s1])
    dsts = jnp.stack([d0, d1])
    zeros = jnp.zeros((NPAD, D), jnp.float32)

    seg = _make_sc_segsum(N, D, CH, NPAD)
    agg, deg = seg(xs, srcs, dsts, zeros)

    RB = 512
    nb = -(-N // RB)
    fin = pl.pallas_call(
        _finalize_body,
        grid=(nb,),
        in_specs=[
            pl.BlockSpec((RB, D), lambda i: (i, 0)),
            pl.BlockSpec((RB, D), lambda i: (i, 0)),
            pl.BlockSpec((NS, RB), lambda i: (0, i)),
            pl.BlockSpec((D, D), lambda i: (0, 0)),
            pl.BlockSpec((1, D), lambda i: (0, 0)),
            pl.BlockSpec((D, D), lambda i: (0, 0)),
        ],
        out_specs=pl.BlockSpec((RB, D), lambda i: (i, 0)),
        out_shape=jax.ShapeDtypeStruct((N, D), jnp.float32),
    )
    bl2 = b_l.reshape(1, D)
    out_src = fin(x_src, agg[0], deg[0], W_l, bl2, W_r)
    out_ref = fin(x_ref, agg[1], deg[1], W_l, bl2, W_r)
    return out_src, out_ref


# speedup vs baseline: 1.4169x; 1.0073x over previous
import functools

import jax
import jax.numpy as jnp
from jax import lax
from jax.experimental import pallas as pl
from jax.experimental.pallas import tpu as pltpu
from jax.experimental.pallas import tpu_sc as plsc

NC = 2
NS = 16
L = 16
CHUNK = 64
RING = 4
IG = 8


@functools.lru_cache(maxsize=None)
def _make_sc_segsum(N, D, CH, NPAD):
    ROWS_PT = NPAD // NS
    NG = CH // IG
    mesh = plsc.VectorSubcoreMesh(core_axis_name="c", subcore_axis_name="s")

    @functools.partial(
        pl.kernel,
        out_type=(
            jax.ShapeDtypeStruct((NC, NPAD, D), jnp.float32),
            jax.ShapeDtypeStruct((NC, NS, NPAD), jnp.float32),
        ),
        mesh=mesh,
        compiler_params=pltpu.CompilerParams(needs_layout_passes=False),
        scratch_types=(
            pltpu.VMEM_SHARED((NPAD, D), jnp.float32),
            pltpu.VMEM((2, IG, CHUNK), jnp.int32),
            pltpu.VMEM((2, IG, CHUNK), jnp.int32),
            pltpu.VMEM((RING, CHUNK, D), jnp.float32),
            pltpu.VMEM((NPAD,), jnp.float32),
            [pltpu.SemaphoreType.DMA] * RING,
            [pltpu.SemaphoreType.DMA] * RING,
            pltpu.SemaphoreType.DMA,
        ),
    )
    def seg(xs_hbm, srcs_hbm, dsts_hbm, zeros_hbm, agg_out, deg_out,
            shared_agg, src_v, dst_v, rows_v, deg_v, sg, ss, si):
        c = lax.axis_index("c")
        s = lax.axis_index("s")
        x_hbm = xs_hbm.at[c]
        pltpu.sync_copy(zeros_hbm.at[pl.ds(s * ROWS_PT, ROWS_PT)],
                        shared_agg.at[pl.ds(s * ROWS_PT, ROWS_PT)])
        zeros16 = jnp.zeros((L,), jnp.float32)

        def zero_body(i, carry):
            deg_v[pl.ds(i * L, L)] = zeros16
            return carry

        lax.fori_loop(0, NPAD // L, zero_body, 0)
        pltpu.sync_copy(srcs_hbm.at[c, s, pl.ds(0, IG)], src_v.at[0])
        pltpu.sync_copy(dsts_hbm.at[c, s, pl.ds(0, IG)], dst_v.at[0])
        plsc.subcore_barrier()

        ones = jnp.full((L,), 1.0, jnp.float32)
        for b in range(3):
            pltpu.async_copy(x_hbm.at[src_v.at[0, b]], rows_v.at[b], sg[b])

        def group_body(g, carry):
            slot = lax.rem(g, 2)
            nslot = lax.rem(g + 1, 2)

            for jj in range(IG):
                j = g * IG + jj
                b = jj % RING
                b3 = (jj + 3) % RING
                if jj == 2:
                    @pl.when(g + 1 < NG)
                    def _():
                        pltpu.async_copy(
                            srcs_hbm.at[c, s, pl.ds((g + 1) * IG, IG)],
                            src_v.at[nslot], si)
                        pltpu.async_copy(
                            dsts_hbm.at[c, s, pl.ds((g + 1) * IG, IG)],
                            dst_v.at[nslot], si)
                if jj == IG - 3:
                    @pl.when(g + 1 < NG)
                    def _():
                        pltpu.make_async_copy(
                            srcs_hbm.at[c, s, pl.ds((g + 1) * IG, IG)],
                            src_v.at[nslot], si).wait()
                        pltpu.make_async_copy(
                            dsts_hbm.at[c, s, pl.ds((g + 1) * IG, IG)],
                            dst_v.at[nslot], si).wait()

                @pl.when(j >= 1)
                def _():
                    if jj >= 1:
                        pidx = dst_v.at[slot, jj - 1]
                    else:
                        pidx = dst_v.at[nslot, IG - 1]
                    pltpu.make_async_copy(
                        rows_v.at[b3], shared_agg.at[pidx], ss[b3]).wait()

                @pl.when(j + 3 < CH)
                def _():
                    if jj + 3 < IG:
                        gidx = src_v.at[slot, jj + 3]
                    else:
                        gidx = src_v.at[nslot, jj + 3 - IG]
                    pltpu.async_copy(x_hbm.at[gidx], rows_v.at[b3], sg[b3])

                pltpu.make_async_copy(
                    x_hbm.at[src_v.at[slot, jj]], rows_v.at[b], sg[b]).wait()
                pltpu.async_copy(
                    rows_v.at[b], shared_agg.at[dst_v.at[slot, jj]], ss[b],
                    add=True)
                for k in range(CHUNK // L):
                    d = dst_v[slot, jj, pl.ds(k * L, L)]
                    plsc.addupdate_scatter(deg_v, [d], ones)
            return carry

        lax.fori_loop(0, NG, group_body, 0)
        lslot = (NG - 1) % 2
        jL = CH - 1
        pltpu.make_async_copy(
            rows_v.at[jL % RING],
            shared_agg.at[dst_v.at[lslot, IG - 1]], ss[jL % RING]).wait()
        plsc.subcore_barrier()
        pltpu.sync_copy(shared_agg.at[pl.ds(s * ROWS_PT, ROWS_PT)],
                        agg_out.at[c, pl.ds(s * ROWS_PT, ROWS_PT)])
        pltpu.sync_copy(deg_v, deg_out.at[c, s])

    return seg


def _finalize_body(x_ref, agg_ref, deg_ref, wl_ref, bl_ref, wr_ref, o_ref):
    deg = jnp.sum(deg_ref[...], axis=0)
    mean = agg_ref[...] / jnp.clip(deg, 1.0)[:, None]
    dn = (((1,), (1,)), ((), ()))
    o_ref[...] = (
        lax.dot_general(mean, wl_ref[...], dn, preferred_element_type=jnp.float32)
        + lax.dot_general(x_ref[...], wr_ref[...], dn, preferred_element_type=jnp.float32)
        + bl_ref[...])


def kernel(x_src, x_ref, src_edge_indices, ref_edge_indices, W_l, b_l, W_r):
    N, D = x_src.shape
    E = src_edge_indices.shape[0]
    CH = -(-E // (NS * CHUNK))
    CH = -(-CH // IG) * IG
    EP = NS * CH * CHUNK
    NPAD = -(-(N + 1) // 128) * 128

    def prep(edges):
        pad = EP - E
        src = jnp.concatenate(
            [edges[:, 0], jnp.zeros((pad,), jnp.int32)]).reshape(NS, CH, CHUNK)
        dst = jnp.concatenate(
            [edges[:, 1], jnp.full((pad,), N, jnp.int32)]).reshape(NS, CH, CHUNK)
        return src, dst

    s0, d0 = prep(src_edge_indices)
    s1, d1 = prep(ref_edge_indices)
    xs = jnp.stack([x_src, x_ref])
    srcs = jnp.stack([s0, s1])
    dsts = jnp.stack([d0, d1])
    zeros = jnp.zeros((NPAD, D), jnp.float32)

    seg = _make_sc_segsum(N, D, CH, NPAD)
    agg, deg = seg(xs, srcs, dsts, zeros)

    RB = 512
    nb = -(-N // RB)
    fin = pl.pallas_call(
        _finalize_body,
        grid=(nb,),
        in_specs=[
            pl.BlockSpec((RB, D), lambda i: (i, 0)),
            pl.BlockSpec((RB, D), lambda i: (i, 0)),
            pl.BlockSpec((NS, RB), lambda i: (0, i)),
            pl.BlockSpec((D, D), lambda i: (0, 0)),
            pl.BlockSpec((1, D), lambda i: (0, 0)),
            pl.BlockSpec((D, D), lambda i: (0, 0)),
        ],
        out_specs=pl.BlockSpec((RB, D), lambda i: (i, 0)),
        out_shape=jax.ShapeDtypeStruct((N, D), jnp.float32),
    )
    bl2 = b_l.reshape(1, D)
    out_src = fin(x_src, agg[0], deg[0], W_l, bl2, W_r)
    out_ref = fin(x_ref, agg[1], deg[1], W_l, bl2, W_r)
    return out_src, out_ref


# speedup vs baseline: 1.4494x; 1.0229x over previous
import functools

import jax
import jax.numpy as jnp
from jax import lax
from jax.experimental import pallas as pl
from jax.experimental.pallas import tpu as pltpu
from jax.experimental.pallas import tpu_sc as plsc

NC = 2
NS = 16
L = 16
CHUNK = 128
RING = 2
IG = 8


@functools.lru_cache(maxsize=None)
def _make_sc_segsum(N, D, CH, NPAD):
    ROWS_PT = NPAD // NS
    NG = CH // IG
    mesh = plsc.VectorSubcoreMesh(core_axis_name="c", subcore_axis_name="s")

    @functools.partial(
        pl.kernel,
        out_type=(
            jax.ShapeDtypeStruct((NC, NPAD, D), jnp.float32),
            jax.ShapeDtypeStruct((NC, NS, NPAD), jnp.float32),
        ),
        mesh=mesh,
        compiler_params=pltpu.CompilerParams(needs_layout_passes=False),
        scratch_types=(
            pltpu.VMEM_SHARED((NPAD, D), jnp.float32),
            pltpu.VMEM((2, IG, CHUNK), jnp.int32),
            pltpu.VMEM((2, IG, CHUNK), jnp.int32),
            pltpu.VMEM((RING, CHUNK, D), jnp.float32),
            pltpu.VMEM((NPAD,), jnp.float32),
            [pltpu.SemaphoreType.DMA] * RING,
            pltpu.SemaphoreType.DMA,
        ),
    )
    def seg(xs_hbm, srcs_hbm, dsts_hbm, zeros_hbm, agg_out, deg_out,
            shared_agg, src_v, dst_v, rows_v, deg_v, sg, si):
        c = lax.axis_index("c")
        s = lax.axis_index("s")
        x_hbm = xs_hbm.at[c]
        pltpu.sync_copy(zeros_hbm.at[pl.ds(s * ROWS_PT, ROWS_PT)],
                        shared_agg.at[pl.ds(s * ROWS_PT, ROWS_PT)])
        zeros16 = jnp.zeros((L,), jnp.float32)

        def zero_body(i, carry):
            deg_v[pl.ds(i * L, L)] = zeros16
            return carry

        lax.fori_loop(0, NPAD // L, zero_body, 0)
        pltpu.sync_copy(srcs_hbm.at[c, s, pl.ds(0, IG)], src_v.at[0])
        pltpu.sync_copy(dsts_hbm.at[c, s, pl.ds(0, IG)], dst_v.at[0])
        plsc.subcore_barrier()

        ones = jnp.full((L,), 1.0, jnp.float32)
        for b in range(RING):
            pltpu.async_copy(x_hbm.at[src_v.at[0, b]], rows_v.at[b], sg[b])

        def group_body(g, carry):
            slot = lax.rem(g, 2)
            nslot = lax.rem(g + 1, 2)

            for jj in range(IG):
                j = g * IG + jj
                b = jj % RING
                if jj == 2:
                    @pl.when(g + 1 < NG)
                    def _():
                        pltpu.async_copy(
                            srcs_hbm.at[c, s, pl.ds((g + 1) * IG, IG)],
                            src_v.at[nslot], si)
                        pltpu.async_copy(
                            dsts_hbm.at[c, s, pl.ds((g + 1) * IG, IG)],
                            dst_v.at[nslot], si)
                if jj == IG - RING:
                    @pl.when(g + 1 < NG)
                    def _():
                        pltpu.make_async_copy(
                            srcs_hbm.at[c, s, pl.ds((g + 1) * IG, IG)],
                            src_v.at[nslot], si).wait()
                        pltpu.make_async_copy(
                            dsts_hbm.at[c, s, pl.ds((g + 1) * IG, IG)],
                            dst_v.at[nslot], si).wait()

                pltpu.make_async_copy(
                    x_hbm.at[src_v.at[slot, jj]], rows_v.at[b], sg[b]).wait()
                pltpu.sync_copy(
                    rows_v.at[b], shared_agg.at[dst_v.at[slot, jj]], add=True)
                for k in range(CHUNK // L):
                    d = dst_v[slot, jj, pl.ds(k * L, L)]
                    plsc.addupdate_scatter(deg_v, [d], ones)

                @pl.when(j + RING < CH)
                def _():
                    if jj + RING < IG:
                        gidx = src_v.at[slot, jj + RING]
                    else:
                        gidx = src_v.at[nslot, jj + RING - IG]
                    pltpu.async_copy(x_hbm.at[gidx], rows_v.at[b], sg[b])
            return carry

        lax.fori_loop(0, NG, group_body, 0)
        plsc.subcore_barrier()
        pltpu.sync_copy(shared_agg.at[pl.ds(s * ROWS_PT, ROWS_PT)],
                        agg_out.at[c, pl.ds(s * ROWS_PT, ROWS_PT)])
        pltpu.sync_copy(deg_v, deg_out.at[c, s])

    return seg


def _finalize_body(x_ref, agg_ref, deg_ref, wl_ref, bl_ref, wr_ref, o_ref):
    deg = jnp.sum(deg_ref[...], axis=0)
    mean = agg_ref[...] / jnp.clip(deg, 1.0)[:, None]
    dn = (((1,), (1,)), ((), ()))
    o_ref[...] = (
        lax.dot_general(mean, wl_ref[...], dn, preferred_element_type=jnp.float32)
        + lax.dot_general(x_ref[...], wr_ref[...], dn, preferred_element_type=jnp.float32)
        + bl_ref[...])


def kernel(x_src, x_ref, src_edge_indices, ref_edge_indices, W_l, b_l, W_r):
    N, D = x_src.shape
    E = src_edge_indices.shape[0]
    CH = -(-E // (NS * CHUNK))
    CH = -(-CH // IG) * IG
    EP = NS * CH * CHUNK
    NPAD = -(-(N + 1) // 128) * 128

    def prep(edges):
        pad = EP - E
        src = jnp.concatenate(
            [edges[:, 0], jnp.zeros((pad,), jnp.int32)]).reshape(NS, CH, CHUNK)
        dst = jnp.concatenate(
            [edges[:, 1], jnp.full((pad,), N, jnp.int32)]).reshape(NS, CH, CHUNK)
        return src, dst

    s0, d0 = prep(src_edge_indices)
    s1, d1 = prep(ref_edge_indices)
    xs = jnp.stack([x_src, x_ref])
    srcs = jnp.stack([s0, s1])
    dsts = jnp.stack([d0, d1])
    zeros = jnp.zeros((NPAD, D), jnp.float32)

    seg = _make_sc_segsum(N, D, CH, NPAD)
    agg, deg = seg(xs, srcs, dsts, zeros)

    RB = 512
    nb = -(-N // RB)
    fin = pl.pallas_call(
        _finalize_body,
        grid=(nb,),
        in_specs=[
            pl.BlockSpec((RB, D), lambda i: (i, 0)),
            pl.BlockSpec((RB, D), lambda i: (i, 0)),
            pl.BlockSpec((NS, RB), lambda i: (0, i)),
            pl.BlockSpec((D, D), lambda i: (0, 0)),
            pl.BlockSpec((1, D), lambda i: (0, 0)),
            pl.BlockSpec((D, D), lambda i: (0, 0)),
        ],
        out_specs=pl.BlockSpec((RB, D), lambda i: (i, 0)),
        out_shape=jax.ShapeDtypeStruct((N, D), jnp.float32),
    )
    bl2 = b_l.reshape(1, D)
    out_src = fin(x_src, agg[0], deg[0], W_l, bl2, W_r)
    out_ref = fin(x_ref, agg[1], deg[1], W_l, bl2, W_r)
    return out_src, out_ref


# speedup vs baseline: 1.5683x; 1.0820x over previous
import functools

import jax
import jax.numpy as jnp
from jax import lax
from jax.experimental import pallas as pl
from jax.experimental.pallas import tpu as pltpu
from jax.experimental.pallas import tpu_sc as plsc

NC = 2
NS = 16
L = 16
CHUNK = 128
RING = 2
IG = 8


@functools.lru_cache(maxsize=None)
def _make_sc_segsum(N, D, CH, NPAD):
    ROWS_PT = NPAD // NS
    NG = CH // IG
    mesh = plsc.VectorSubcoreMesh(core_axis_name="c", subcore_axis_name="s")

    @functools.partial(
        pl.kernel,
        out_type=(
            jax.ShapeDtypeStruct((NC, NPAD, D), jnp.float32),
            jax.ShapeDtypeStruct((NC, NS, NPAD), jnp.float32),
        ),
        mesh=mesh,
        compiler_params=pltpu.CompilerParams(needs_layout_passes=False),
        scratch_types=(
            pltpu.VMEM_SHARED((NPAD, D), jnp.float32),
            pltpu.VMEM((2, IG, CHUNK), jnp.int32),
            pltpu.VMEM((2, IG, CHUNK), jnp.int32),
            pltpu.VMEM((RING, CHUNK, D), jnp.float32),
            pltpu.VMEM((NPAD,), jnp.float32),
            [pltpu.SemaphoreType.DMA] * RING,
            pltpu.SemaphoreType.DMA,
        ),
    )
    def seg(xs_hbm, srcs_hbm, dsts_hbm, zeros_hbm, agg_out, deg_out,
            shared_agg, src_v, dst_v, rows_v, deg_v, sg, si):
        c = lax.axis_index("c")
        s = lax.axis_index("s")
        x_hbm = xs_hbm.at[c]
        pltpu.sync_copy(zeros_hbm,
                        shared_agg.at[pl.ds(s * ROWS_PT, ROWS_PT)])
        zeros16 = jnp.zeros((L,), jnp.float32)

        def zero_body(i, carry):
            deg_v[pl.ds(i * L, L)] = zeros16
            return carry

        lax.fori_loop(0, NPAD // L, zero_body, 0)
        pltpu.sync_copy(srcs_hbm.at[c, s, pl.ds(0, IG)], src_v.at[0])
        pltpu.sync_copy(dsts_hbm.at[c, s, pl.ds(0, IG)], dst_v.at[0])
        plsc.subcore_barrier()

        ones = jnp.full((L,), 1.0, jnp.float32)
        for b in range(RING):
            pltpu.async_copy(x_hbm.at[src_v.at[0, b]], rows_v.at[b], sg[b])

        def group_body(g, carry):
            slot = lax.rem(g, 2)
            nslot = lax.rem(g + 1, 2)

            for jj in range(IG):
                j = g * IG + jj
                b = jj % RING
                if jj == 2:
                    @pl.when(g + 1 < NG)
                    def _():
                        pltpu.async_copy(
                            srcs_hbm.at[c, s, pl.ds((g + 1) * IG, IG)],
                            src_v.at[nslot], si)
                        pltpu.async_copy(
                            dsts_hbm.at[c, s, pl.ds((g + 1) * IG, IG)],
                            dst_v.at[nslot], si)
                if jj == IG - RING:
                    @pl.when(g + 1 < NG)
                    def _():
                        pltpu.make_async_copy(
                            srcs_hbm.at[c, s, pl.ds((g + 1) * IG, IG)],
                            src_v.at[nslot], si).wait()
                        pltpu.make_async_copy(
                            dsts_hbm.at[c, s, pl.ds((g + 1) * IG, IG)],
                            dst_v.at[nslot], si).wait()

                pltpu.make_async_copy(
                    x_hbm.at[src_v.at[slot, jj]], rows_v.at[b], sg[b]).wait()
                pltpu.sync_copy(
                    rows_v.at[b], shared_agg.at[dst_v.at[slot, jj]], add=True)
                for k in range(CHUNK // L):
                    d = dst_v[slot, jj, pl.ds(k * L, L)]
                    plsc.addupdate_scatter(deg_v, [d], ones)

                @pl.when(j + RING < CH)
                def _():
                    if jj + RING < IG:
                        gidx = src_v.at[slot, jj + RING]
                    else:
                        gidx = src_v.at[nslot, jj + RING - IG]
                    pltpu.async_copy(x_hbm.at[gidx], rows_v.at[b], sg[b])
            return carry

        lax.fori_loop(0, NG, group_body, 0)
        plsc.subcore_barrier()
        pltpu.sync_copy(shared_agg.at[pl.ds(s * ROWS_PT, ROWS_PT)],
                        agg_out.at[c, pl.ds(s * ROWS_PT, ROWS_PT)])
        pltpu.sync_copy(deg_v, deg_out.at[c, s])

    return seg


def _finalize_body(x_ref, agg_ref, deg_ref, wl_ref, bl_ref, wr_ref, o_ref):
    deg = jnp.sum(deg_ref[0], axis=0)
    mean = agg_ref[0] / jnp.clip(deg, 1.0)[:, None]
    dn = (((1,), (1,)), ((), ()))
    o_ref[0] = (
        lax.dot_general(mean, wl_ref[...], dn, preferred_element_type=jnp.float32)
        + lax.dot_general(x_ref[0], wr_ref[...], dn, preferred_element_type=jnp.float32)
        + bl_ref[...])


def kernel(x_src, x_ref, src_edge_indices, ref_edge_indices, W_l, b_l, W_r):
    N, D = x_src.shape
    E = src_edge_indices.shape[0]
    CH = -(-E // (NS * CHUNK))
    CH = -(-CH // IG) * IG
    EP = NS * CH * CHUNK
    NPAD = -(-(N + 1) // 128) * 128

    pad = EP - E

    def prep(col, fill):
        parts = []
        for e in (src_edge_indices, ref_edge_indices):
            parts.append(e[:, col])
            if pad:
                parts.append(jnp.full((pad,), fill, jnp.int32))
        return jnp.concatenate(parts).reshape(2, NS, CH, CHUNK)

    srcs = prep(0, 0)
    dsts = prep(1, N)
    xs = jnp.stack([x_src, x_ref])
    zeros = jnp.zeros((NPAD // NS, D), jnp.float32)

    seg = _make_sc_segsum(N, D, CH, NPAD)
    agg, deg = seg(xs, srcs, dsts, zeros)

    RB = 512
    nb = -(-N // RB)
    outs = pl.pallas_call(
        _finalize_body,
        grid=(2, nb),
        in_specs=[
            pl.BlockSpec((1, RB, D), lambda g, i: (g, i, 0)),
            pl.BlockSpec((1, RB, D), lambda g, i: (g, i, 0)),
            pl.BlockSpec((1, NS, RB), lambda g, i: (g, 0, i)),
            pl.BlockSpec((D, D), lambda g, i: (0, 0)),
            pl.BlockSpec((1, D), lambda g, i: (0, 0)),
            pl.BlockSpec((D, D), lambda g, i: (0, 0)),
        ],
        out_specs=pl.BlockSpec((1, RB, D), lambda g, i: (g, i, 0)),
        out_shape=jax.ShapeDtypeStruct((2, N, D), jnp.float32),
    )(xs, agg, deg, W_l, b_l.reshape(1, D), W_r)
    return outs[0], outs[1]
